# baseline (speedup 1.0000x reference)
import functools

import jax
import jax.numpy as jnp
from jax.experimental import pallas as pl
from jax.experimental.pallas import tpu as pltpu
from jax.experimental.pallas import tpu_sc as plsc

N = 10000
E = 320000
INPLANES = 128
PLANES = 32
EXPANSION = 4
EPS = 1e-5

NC = 2
NS = 16
NW = NC * NS
K = 128
CH = 80
EPW = CH * K
EPAD = NW * EPW
NB = 8
GRP = CH // NB
Q16 = K // 16
ZR = 640
NACC = NS * ZR
MR = NACC // 16
IC = MR // K

G = 5
R = N // G

_PREC = jax.lax.Precision.DEFAULT


def _dot(a, b):
    return jnp.dot(a, b, preferred_element_type=jnp.float32, precision=_PREC)


def _leaky(h):
    return jnp.where(h >= 0, h, 0.1 * h)


def _norm_from_stats(o, stats, g, b):
    mu = stats[0:1, :] * (1.0 / N)
    var = stats[1:2, :] * (1.0 / N) - mu * mu
    return g * (o - mu) / jnp.sqrt(var + EPS) + b


def _accum_stats(i, o, stats_ref):
    @pl.when(i == 0)
    def _():
        stats_ref[...] = jnp.zeros_like(stats_ref)
    s = jnp.sum(o, axis=0, keepdims=True)
    sq = jnp.sum(o * o, axis=0, keepdims=True)
    stats_ref[...] += jnp.concatenate([s, sq], axis=0)


def _rowspec(w):
    return pl.BlockSpec((R, w), lambda i: (i, 0))


def _fullspec(h, w):
    return pl.BlockSpec((h, w), lambda i: (0, 0))


def _stage_a1(x_ref, m_ref, w1_ref, b1_ref, o_ref, stats_ref):
    i = pl.program_id(0)
    m = m_ref[...]
    hold = (m > 0).astype(jnp.float32)
    ratio = hold / jnp.clip(m, EPS, None)
    o = _dot(x_ref[...] * m, w1_ref[...])
    o = o * ratio + b1_ref[...] * hold
    o_ref[...] = o
    _accum_stats(i, o, stats_ref)


_stage_a1_call = pl.pallas_call(
    _stage_a1,
    grid=(G,),
    in_specs=[_rowspec(INPLANES), _rowspec(1), _fullspec(INPLANES, PLANES),
              _fullspec(1, PLANES)],
    out_specs=(_rowspec(PLANES), _fullspec(2, PLANES)),
    out_shape=(
        jax.ShapeDtypeStruct((N, PLANES), jnp.float32),
        jax.ShapeDtypeStruct((2, PLANES), jnp.float32),
    ),
)


def _stage_a2(o_ref, m_ref, stats_ref, g1_ref, be1_ref, pay_ref, m1_ref):
    m = m_ref[...]
    hold = (m > 0).astype(jnp.float32)
    h = _leaky(_norm_from_stats(o_ref[...], stats_ref[...],
                                g1_ref[...], be1_ref[...]))
    pay_ref[...] = h * hold
    m1_ref[...] = hold


_stage_a2_call = pl.pallas_call(
    _stage_a2,
    grid=(G,),
    in_specs=[_rowspec(PLANES), _rowspec(1), _fullspec(2, PLANES),
              _fullspec(1, PLANES), _fullspec(1, PLANES)],
    out_specs=(_rowspec(PLANES), _rowspec(1)),
    out_shape=(
        jax.ShapeDtypeStruct((N, PLANES), jnp.float32),
        jax.ShapeDtypeStruct((N, 1), jnp.float32),
    ),
)


@functools.cache
def _sc_segsum_call():
    mesh = plsc.VectorSubcoreMesh(
        core_axis_name="c", subcore_axis_name="s",
        num_cores=NC, num_subcores=NS)

    @functools.partial(
        pl.kernel,
        out_type=(
            jax.ShapeDtypeStruct((NC, NS, ZR, PLANES), jnp.float32),
            jax.ShapeDtypeStruct((NC, 2, MR, 16), jnp.float32),
        ),
        mesh=mesh,
        compiler_params=pltpu.CompilerParams(
            use_tc_tiling_on_sc=False, needs_layout_passes=False),
        scratch_types=[
            pltpu.VMEM((CH, K), jnp.int32),
            pltpu.VMEM((CH, K), jnp.int32),
            pltpu.VMEM((NB, K, PLANES), jnp.float32),
            pltpu.VMEM((N,), jnp.float32),
            pltpu.VMEM((MR, 16), jnp.float32),
            pltpu.VMEM((MR, 16), jnp.float32),
            pltpu.VMEM((IC, K), jnp.int32),
            pltpu.VMEM_SHARED((NACC, PLANES), jnp.float32),
            pltpu.VMEM_SHARED((MR, 16), jnp.float32),
            pltpu.VMEM_SHARED((MR, 16), jnp.float32),
            pltpu.SemaphoreType.DMA((NB,)),
            pltpu.SemaphoreType.DMA((NB,)),
        ],
    )
    def _sc_segsum(payload_hbm, m1_hbm, src_hbm, dst_hbm, zeros_hbm,
                   zeros_mc_hbm, iota_hbm, out_hbm, mc_hbm,
                   src_v, dst_v, bufs, m1_v, ms_v, ct_v, iota_v,
                   acc_sh, ms_sh, ct_sh, gsem, ssem):
        c = jax.lax.axis_index("c")
        s = jax.lax.axis_index("s")
        wid = c * NS + s
        pltpu.sync_copy(zeros_hbm, acc_sh.at[pl.ds(s * ZR, ZR)])
        @pl.when(s == 0)
        def _():
            pltpu.sync_copy(zeros_mc_hbm, ms_sh)
            pltpu.sync_copy(zeros_mc_hbm, ct_sh)
        pltpu.sync_copy(src_hbm.at[wid], src_v)
        pltpu.sync_copy(dst_hbm.at[wid], dst_v)
        pltpu.sync_copy(iota_hbm, iota_v)
        pltpu.sync_copy(m1_hbm, m1_v)
        zer16 = jnp.zeros((16,), jnp.float32)

        def zbody(q, carry):
            ms_v[q] = zer16
            ct_v[q] = zer16
            return carry

        jax.lax.fori_loop(0, MR, zbody, 0)
        plsc.subcore_barrier()

        one16 = jnp.ones((16,), jnp.float32)

        def group(g, carry):
            base = g * NB
            for b in range(NB):
                pltpu.async_copy(payload_hbm.at[src_v.at[base + b]],
                                 bufs.at[b], gsem.at[b])

            def mbody(q, carry2):
                j = base + q // Q16
                o16 = (q % Q16) * 16
                sv = src_v[j, pl.ds(o16, 16)]
                dv = dst_v[j, pl.ds(o16, 16)]
                dr = jax.lax.shift_right_logical(dv, 4)
                dl = jax.lax.bitwise_and(dv, 15)
                mv = plsc.load_gather(m1_v, [sv])
                plsc.addupdate_scatter(ms_v, [dr, dl], mv)
                plsc.addupdate_scatter(ct_v, [dr, dl], one16)
                return carry2

            jax.lax.fori_loop(0, NB * Q16, mbody, 0)

            for b in range(NB):
                pltpu.make_async_copy(payload_hbm.at[src_v.at[base + b]],
                                      bufs.at[b], gsem.at[b]).wait()
                pltpu.async_copy(bufs.at[b], acc_sh.at[dst_v.at[base + b]],
                                 ssem.at[b], add=True)
            for b in range(NB):
                pltpu.make_async_copy(bufs.at[b],
                                      acc_sh.at[dst_v.at[base + b]],
                                      ssem.at[b]).wait()
            return carry

        jax.lax.fori_loop(0, GRP, group, 0)

        for p in range(IC):
            pltpu.sync_copy(ms_v.at[pl.ds(p * K, K)],
                            ms_sh.at[iota_v.at[p]], add=True)
            pltpu.sync_copy(ct_v.at[pl.ds(p * K, K)],
                            ct_sh.at[iota_v.at[p]], add=True)
        plsc.subcore_barrier()
        pltpu.sync_copy(acc_sh.at[pl.ds(s * ZR, ZR)], out_hbm.at[c, s])
        @pl.when(s == 0)
        def _():
            pltpu.sync_copy(ms_sh, mc_hbm.at[c, 0])
            pltpu.sync_copy(ct_sh, mc_hbm.at[c, 1])

    return _sc_segsum


def _stage_c1(acc_ref, p_ref, msc_ref, m_ref, w2n_ref, w2s_ref, b2_ref,
              o2_ref, mout_ref, stats_ref):
    i = pl.program_id(0)
    p = p_ref[...]
    t = acc_ref[0] + acc_ref[1] + p
    m = m_ref[...]
    hold1 = (m > 0).astype(jnp.float32)
    msc = msc_ref[...]
    msum = msc[:, 0:1] + msc[:, 1:2] + hold1
    cnt = msc[:, 2:3] + msc[:, 3:4] + 1.0
    hold2 = (msum > 0).astype(jnp.float32)
    ratio2 = hold2 * cnt / jnp.clip(msum, EPS, None)
    o2 = _dot(t, w2n_ref[...]) + _dot(p, w2s_ref[...])
    o2 = o2 * ratio2 + b2_ref[...] * hold2
    o2_ref[...] = o2
    mout_ref[...] = jnp.clip(msum, 0.0, 1.0)
    _accum_stats(i, o2, stats_ref)


_stage_c1_call = pl.pallas_call(
    _stage_c1,
    grid=(G,),
    in_specs=[pl.BlockSpec((NC, R, PLANES), lambda i: (0, i, 0)),
              _rowspec(PLANES), _rowspec(4), _rowspec(1),
              _fullspec(PLANES, PLANES), _fullspec(PLANES, PLANES),
              _fullspec(1, PLANES)],
    out_specs=(_rowspec(PLANES), _rowspec(1), _fullspec(2, PLANES)),
    out_shape=(
        jax.ShapeDtypeStruct((N, PLANES), jnp.float32),
        jax.ShapeDtypeStruct((N, 1), jnp.float32),
        jax.ShapeDtypeStruct((2, PLANES), jnp.float32),
    ),
)


def _stage_c2(o2_ref, mout_ref, stats_ref, g2_ref, be2_ref, w3_ref, b3_ref,
              o3_ref, stats3_ref):
    i = pl.program_id(0)
    h2 = _leaky(_norm_from_stats(o2_ref[...], stats_ref[...],
                                 g2_ref[...], be2_ref[...]))
    mout = mout_ref[...]
    hold3 = (mout > 0).astype(jnp.float32)
    ratio3 = hold3 / jnp.clip(mout, EPS, None)
    o3 = _dot(h2 * mout, w3_ref[...]) * ratio3 + b3_ref[...] * hold3
    o3_ref[...] = o3
    _accum_stats(i, o3, stats3_ref)


_stage_c2_call = pl.pallas_call(
    _stage_c2,
    grid=(G,),
    in_specs=[_rowspec(PLANES), _rowspec(1), _fullspec(2, PLANES),
              _fullspec(1, PLANES), _fullspec(1, PLANES),
              _fullspec(PLANES, INPLANES), _fullspec(1, INPLANES)],
    out_specs=(_rowspec(INPLANES), _fullspec(2, INPLANES)),
    out_shape=(
        jax.ShapeDtypeStruct((N, INPLANES), jnp.float32),
        jax.ShapeDtypeStruct((2, INPLANES), jnp.float32),
    ),
)


def _stage_c3(o3_ref, x_ref, m_ref, mout_ref, stats_ref, g3_ref, be3_ref,
              out_ref, omask_ref):
    h3 = _norm_from_stats(o3_ref[...], stats_ref[...],
                          g3_ref[...], be3_ref[...])
    out_ref[...] = _leaky(h3 + x_ref[...])
    omask_ref[...] = jnp.clip(mout_ref[...] + m_ref[...], 0.0, 1.0)


_stage_c3_call = pl.pallas_call(
    _stage_c3,
    grid=(G,),
    in_specs=[_rowspec(INPLANES), _rowspec(INPLANES), _rowspec(1),
              _rowspec(1), _fullspec(2, INPLANES), _fullspec(1, INPLANES),
              _fullspec(1, INPLANES)],
    out_specs=(_rowspec(INPLANES), _rowspec(1)),
    out_shape=(
        jax.ShapeDtypeStruct((N, INPLANES), jnp.float32),
        jax.ShapeDtypeStruct((N, 1), jnp.float32),
    ),
)


def kernel(x, mask, edge_index, W1, b1, g1, be1, W2s, W2n, b2, g2, be2,
           W3, b3, g3, be3):
    npad = EPAD - E
    src = jnp.concatenate(
        [edge_index[0], jnp.zeros((npad,), jnp.int32)]).reshape(NW, CH, K)
    dst = jnp.concatenate(
        [edge_index[1], jnp.full((npad,), N, jnp.int32)]).reshape(NW, CH, K)

    o1, stats1 = _stage_a1_call(x, mask, W1, b1.reshape(1, PLANES))
    payload, m1 = _stage_a2_call(o1, mask, stats1, g1.reshape(1, PLANES),
                                 be1.reshape(1, PLANES))
    zeros_blk = jnp.zeros((ZR, PLANES), jnp.float32)
    zeros_mc = jnp.zeros((MR, 16), jnp.float32)
    iota_rows = jnp.arange(MR, dtype=jnp.int32).reshape(IC, K)
    acc, mc = _sc_segsum_call()(payload, m1.reshape(N), src, dst, zeros_blk,
                                zeros_mc, iota_rows)
    acc = acc.reshape(NC, NACC, PLANES)
    msc = jnp.transpose(mc.reshape(NC, 2, NACC), (2, 1, 0)).reshape(
        NACC, 2 * NC)
    o2, mout, stats2 = _stage_c1_call(acc, payload, msc, mask,
                                      W2n, W2s, b2.reshape(1, PLANES))
    o3, stats3 = _stage_c2_call(o2, mout, stats2, g2.reshape(1, PLANES),
                                be2.reshape(1, PLANES), W3,
                                b3.reshape(1, INPLANES))
    out, omask = _stage_c3_call(o3, x, mask, mout, stats3,
                                g3.reshape(1, INPLANES),
                                be3.reshape(1, INPLANES))
    return (out, omask)


# speedup vs baseline: 1.5817x; 1.5817x over previous
import functools

import jax
import jax.numpy as jnp
from jax.experimental import pallas as pl
from jax.experimental.pallas import tpu as pltpu
from jax.experimental.pallas import tpu_sc as plsc

N = 10000
E = 320000
INPLANES = 128
PLANES = 32
EXPANSION = 4
EPS = 1e-5

PW = 48
NC = 2
NS = 16
NW = NC * NS
EPW = E // NW
K = 100
CH = EPW // K
NB = 10
ZR = N // NS

G = 5
R = N // G

_HIGH = jax.lax.Precision.DEFAULT


def _dot(a, b):
    return jnp.dot(a, b, preferred_element_type=jnp.float32, precision=_HIGH)


def _leaky(h):
    return jnp.where(h >= 0, h, 0.1 * h)


def _norm_from_stats(o, stats, g, b):
    mu = stats[0:1, :] * (1.0 / N)
    var = stats[1:2, :] * (1.0 / N) - mu * mu
    return g * (o - mu) / jnp.sqrt(var + EPS) + b


def _accum_stats(i, o, stats_ref):
    @pl.when(i == 0)
    def _():
        stats_ref[...] = jnp.zeros_like(stats_ref)
    s = jnp.sum(o, axis=0, keepdims=True)
    sq = jnp.sum(o * o, axis=0, keepdims=True)
    stats_ref[...] += jnp.concatenate([s, sq], axis=0)


def _rowspec(w):
    return pl.BlockSpec((R, w), lambda i: (i, 0))


def _fullspec(h, w):
    return pl.BlockSpec((h, w), lambda i: (0, 0))


def _stage_a1(x_ref, m_ref, w1_ref, b1_ref, o_ref, stats_ref):
    i = pl.program_id(0)
    m = m_ref[...]
    hold = (m > 0).astype(jnp.float32)
    ratio = hold / jnp.clip(m, EPS, None)
    o = _dot(x_ref[...] * m, w1_ref[...])
    o = o * ratio + b1_ref[...] * hold
    ci = jax.lax.broadcasted_iota(jnp.int32, (R, PW), 1)
    o = jnp.where(ci == PLANES, jnp.broadcast_to(m, (R, PW)), o)
    o_ref[...] = o
    _accum_stats(i, o, stats_ref)


_stage_a1_call = pl.pallas_call(
    _stage_a1,
    grid=(G,),
    in_specs=[_rowspec(INPLANES), _rowspec(1), _fullspec(INPLANES, PW),
              _fullspec(1, PW)],
    out_specs=(_rowspec(PW), _fullspec(2, PW)),
    out_shape=(
        jax.ShapeDtypeStruct((N, PW), jnp.float32),
        jax.ShapeDtypeStruct((2, PW), jnp.float32),
    ),
)


def _stage_a2(o_ref, stats_ref, g1_ref, be1_ref, pay_ref):
    ov = o_ref[...]
    m = ov[:, PLANES:PLANES + 1]
    hold = (m > 0).astype(jnp.float32)
    h = _leaky(_norm_from_stats(ov, stats_ref[...],
                                g1_ref[...], be1_ref[...]))
    hm = h * hold
    ci = jax.lax.broadcasted_iota(jnp.int32, (R, PW), 1)
    holdb = jnp.broadcast_to(hold, (R, PW))
    mb = jnp.broadcast_to(m, (R, PW))
    pay_ref[...] = jnp.where(ci < PLANES, hm,
                             jnp.where(ci == PLANES, holdb,
                                       jnp.where(ci == PLANES + 1, 1.0,
                                                 jnp.where(ci == PLANES + 2,
                                                           mb, 0.0))))


_stage_a2_call = pl.pallas_call(
    _stage_a2,
    grid=(G,),
    in_specs=[_rowspec(PW), _fullspec(2, PW), _fullspec(1, PW),
              _fullspec(1, PW)],
    out_specs=_rowspec(PW),
    out_shape=jax.ShapeDtypeStruct((N, PW), jnp.float32),
)


@functools.cache
def _sc_segsum_call():
    mesh = plsc.VectorSubcoreMesh(
        core_axis_name="c", subcore_axis_name="s",
        num_cores=NC, num_subcores=NS)

    @functools.partial(
        pl.kernel,
        out_type=jax.ShapeDtypeStruct((NC, NS, ZR, PW), jnp.float32),
        mesh=mesh,
        compiler_params=pltpu.CompilerParams(use_tc_tiling_on_sc=False),
        scratch_types=[
            pltpu.VMEM((CH, K), jnp.int32),
            pltpu.VMEM((CH, K), jnp.int32),
            pltpu.VMEM((NB, K, PW), jnp.float32),
            pltpu.VMEM_SHARED((N, PW), jnp.float32),
            pltpu.SemaphoreType.DMA((NB,)),
            pltpu.SemaphoreType.DMA((NB,)),
        ],
    )
    def _sc_segsum(payload_hbm, src_hbm, dst_hbm, zeros_hbm, out_hbm,
                   src_v, dst_v, bufs, acc_sh, gsem, ssem):
        c = jax.lax.axis_index("c")
        s = jax.lax.axis_index("s")
        wid = c * NS + s
        pltpu.sync_copy(zeros_hbm, acc_sh.at[pl.ds(s * ZR, ZR)])
        pltpu.sync_copy(src_hbm.at[wid], src_v)
        pltpu.sync_copy(dst_hbm.at[wid], dst_v)
        plsc.subcore_barrier()

        def group(g, carry):
            base = g * NB
            for b in range(NB):
                pltpu.async_copy(payload_hbm.at[src_v.at[base + b]],
                                 bufs.at[b], gsem.at[b])
            for b in range(NB):
                pltpu.make_async_copy(payload_hbm.at[src_v.at[base + b]],
                                      bufs.at[b], gsem.at[b]).wait()
                pltpu.async_copy(bufs.at[b], acc_sh.at[dst_v.at[base + b]],
                                 ssem.at[b], add=True)
            for b in range(NB):
                pltpu.make_async_copy(bufs.at[b],
                                      acc_sh.at[dst_v.at[base + b]],
                                      ssem.at[b]).wait()
            return carry

        jax.lax.fori_loop(0, CH // NB, group, 0)
        plsc.subcore_barrier()
        pltpu.sync_copy(acc_sh.at[pl.ds(s * ZR, ZR)], out_hbm.at[c, s])

    return _sc_segsum


def _stage_c1(acc_ref, p_ref, w2n_ref, w2s_ref, b2_ref,
              o2_ref, omask_ref, stats_ref):
    i = pl.program_id(0)
    p = p_ref[...]
    t = acc_ref[0] + acc_ref[1] + p
    msum = t[:, PLANES:PLANES + 1]
    cnt = t[:, PLANES + 1:PLANES + 2]
    mask = p[:, PLANES + 2:PLANES + 3]
    hold2 = (msum > 0).astype(jnp.float32)
    ratio2 = hold2 * cnt / jnp.clip(msum, EPS, None)
    o2 = _dot(t, w2n_ref[...]) + _dot(p, w2s_ref[...])
    o2 = o2 * ratio2 + b2_ref[...] * hold2
    mout = jnp.clip(msum, 0.0, 1.0)
    ci = jax.lax.broadcasted_iota(jnp.int32, (R, PW), 1)
    o2 = jnp.where(ci == PLANES, jnp.broadcast_to(mout, (R, PW)), o2)
    o2_ref[...] = o2
    omask_ref[...] = jnp.clip(mout + mask, 0.0, 1.0)
    _accum_stats(i, o2, stats_ref)


_stage_c1_call = pl.pallas_call(
    _stage_c1,
    grid=(G,),
    in_specs=[pl.BlockSpec((NC, R, PW), lambda i: (0, i, 0)), _rowspec(PW),
              _fullspec(PW, PW), _fullspec(PW, PW),
              _fullspec(1, PW)],
    out_specs=(_rowspec(PW), _rowspec(1), _fullspec(2, PW)),
    out_shape=(
        jax.ShapeDtypeStruct((N, PW), jnp.float32),
        jax.ShapeDtypeStruct((N, 1), jnp.float32),
        jax.ShapeDtypeStruct((2, PW), jnp.float32),
    ),
)


def _stage_c2(o2_ref, stats_ref, g2_ref, be2_ref, w3_ref, b3_ref,
              o3_ref, stats3_ref):
    i = pl.program_id(0)
    o2v = o2_ref[...]
    mout = o2v[:, PLANES:PLANES + 1]
    h2 = _leaky(_norm_from_stats(o2v, stats_ref[...],
                                 g2_ref[...], be2_ref[...]))
    hold3 = (mout > 0).astype(jnp.float32)
    ratio3 = hold3 / jnp.clip(mout, EPS, None)
    o3 = _dot(h2 * mout, w3_ref[...]) * ratio3 + b3_ref[...] * hold3
    o3_ref[...] = o3
    _accum_stats(i, o3, stats3_ref)


_stage_c2_call = pl.pallas_call(
    _stage_c2,
    grid=(G,),
    in_specs=[_rowspec(PW), _fullspec(2, PW),
              _fullspec(1, PW), _fullspec(1, PW),
              _fullspec(PW, INPLANES), _fullspec(1, INPLANES)],
    out_specs=(_rowspec(INPLANES), _fullspec(2, INPLANES)),
    out_shape=(
        jax.ShapeDtypeStruct((N, INPLANES), jnp.float32),
        jax.ShapeDtypeStruct((2, INPLANES), jnp.float32),
    ),
)


def _stage_c3(o3_ref, x_ref, stats_ref, g3_ref, be3_ref, out_ref):
    h3 = _norm_from_stats(o3_ref[...], stats_ref[...],
                          g3_ref[...], be3_ref[...])
    out_ref[...] = _leaky(h3 + x_ref[...])


_stage_c3_call = pl.pallas_call(
    _stage_c3,
    grid=(G,),
    in_specs=[_rowspec(INPLANES), _rowspec(INPLANES),
              _fullspec(2, INPLANES), _fullspec(1, INPLANES),
              _fullspec(1, INPLANES)],
    out_specs=_rowspec(INPLANES),
    out_shape=jax.ShapeDtypeStruct((N, INPLANES), jnp.float32),
)


def kernel(x, mask, edge_index, W1, b1, g1, be1, W2s, W2n, b2, g2, be2,
           W3, b3, g3, be3):
    src = edge_index[0].reshape(NW, CH, K)
    dst = edge_index[1].reshape(NW, CH, K)
    pd = PW - PLANES
    w1p = jnp.pad(W1, ((0, 0), (0, pd)))
    b1p = jnp.pad(b1, (0, pd)).reshape(1, PW)
    g1p = jnp.pad(g1, (0, pd)).reshape(1, PW)
    be1p = jnp.pad(be1, (0, pd)).reshape(1, PW)
    w2np = jnp.pad(W2n, ((0, pd), (0, pd)))
    w2sp = jnp.pad(W2s, ((0, pd), (0, pd)))
    b2p = jnp.pad(b2, (0, pd)).reshape(1, PW)
    g2p = jnp.pad(g2, (0, pd)).reshape(1, PW)
    be2p = jnp.pad(be2, (0, pd)).reshape(1, PW)
    w3p = jnp.pad(W3, ((0, pd), (0, 0)))

    o1, stats1 = _stage_a1_call(x, mask, w1p, b1p)
    payload = _stage_a2_call(o1, stats1, g1p, be1p)
    zeros_blk = jnp.zeros((ZR, PW), jnp.float32)
    acc = _sc_segsum_call()(payload, src, dst, zeros_blk)
    acc = acc.reshape(NC, N, PW)
    o2, omask, stats2 = _stage_c1_call(acc, payload, w2np, w2sp, b2p)
    o3, stats3 = _stage_c2_call(o2, stats2, g2p, be2p, w3p,
                                b3.reshape(1, INPLANES))
    out = _stage_c3_call(o3, x, stats3, g3.reshape(1, INPLANES),
                         be3.reshape(1, INPLANES))
    return (out, omask)
